# Initial kernel scaffold; baseline (speedup 1.0000x reference)
import jax
import jax.numpy as jnp
from jax.experimental import pallas as pl


def kernel(x, gain, bias, log_mix, log_alpha, log_scale):
    raise NotImplementedError("write your pallas kernel here")


# speedup vs baseline: 7.9116x; 7.9116x over previous
import functools
import jax
import jax.numpy as jnp
from jax.experimental import pallas as pl
from jax.experimental.pallas import tpu as pltpu

K_SIM = 8
K_CON = 4
NEG_BIG = -1.0e9
POS_BIG = 1.0e9
INVALID_THRESH = -0.5e9


def _norm_body(x_ref, xn_ref):
    x = x_ref[...]
    n = jnp.sqrt(jnp.sum(x * x, axis=-1, keepdims=True))
    xn_ref[...] = x / jnp.maximum(n, 1e-12)


def _picks_body(params_ref, xn_ref, idx_ref, w_ref, *, T, Bq, k_sim, k_con):
    qi = pl.program_id(1)
    qbase = qi * Bq
    Xn = xn_ref[0]
    Xnq = xn_ref[0, pl.ds(qbase, Bq), :]
    sim = jax.lax.dot_general(
        Xnq.astype(jnp.bfloat16), Xn.astype(jnp.bfloat16),
        (((1,), (1,)), ((), ())),
        preferred_element_type=jnp.float32,
    )
    rows = qbase + jax.lax.broadcasted_iota(jnp.int32, (Bq, 1), 0)
    cols = jax.lax.broadcasted_iota(jnp.int32, (Bq, T), 1)
    valid = cols < rows
    work = jnp.where(valid, sim, NEG_BIG)

    alpha = params_ref[1]

    idxs = []
    oks = []
    for _ in range(k_sim):
        m = jnp.max(work, axis=1, keepdims=True)
        hit = work == m
        idx = jnp.min(jnp.where(hit, cols, T), axis=1, keepdims=True)
        pick = cols == idx
        ok = m > INVALID_THRESH
        idxs.append(idx)
        oks.append(ok)
        work = jnp.where(pick, 3.0 * NEG_BIG, work)

    deg_sim = functools.reduce(
        lambda a, b: a + b, [ok.astype(jnp.float32) for ok in oks[:k_sim]]
    ) if k_sim else jnp.zeros((Bq, 1), jnp.float32)

    simc = jnp.where(work > INVALID_THRESH, work, POS_BIG)
    mcon = jnp.maximum(0, k_con - (T - rows))
    oks_c = []
    for j in range(k_con):
        mn = jnp.min(simc, axis=1, keepdims=True)
        hit = simc == mn
        idx = jnp.min(jnp.where(hit, cols, T), axis=1, keepdims=True)
        pick = cols == idx
        ok = (mn < -INVALID_THRESH) & (j < mcon)
        idxs.append(idx)
        oks_c.append(ok)
        simc = jnp.where(pick, POS_BIG, simc)

    deg_con = functools.reduce(
        lambda a, b: a + b, [ok.astype(jnp.float32) for ok in oks_c]
    ) if k_con else jnp.zeros((Bq, 1), jnp.float32)

    w_sim = alpha / jnp.maximum(deg_sim, 1.0)
    w_con = (1.0 - alpha) / jnp.maximum(deg_con, 1.0)
    ws = [jnp.where(ok, w_sim, 0.0) for ok in oks]
    ws += [jnp.where(ok, w_con, 0.0) for ok in oks_c]

    idx_ref[0] = jnp.concatenate(idxs, axis=1)
    w_ref[0] = jnp.concatenate(ws, axis=1)


def _agg_body(params_ref, gain_ref, bias_ref, x_ref, idx_ref, w_ref, out_ref,
              *, T, Bq, kp):
    qi = pl.program_id(1)
    qbase = qi * Bq
    X = x_ref[0]
    idx = idx_ref[0]
    w = w_ref[0]
    cols = jax.lax.broadcasted_iota(jnp.int32, (Bq, T), 1)
    A = jnp.zeros((Bq, T), jnp.float32)
    for k in range(kp):
        A = A + jnp.where(cols == idx[:, k:k + 1], w[:, k:k + 1], 0.0)
    ctx = jax.lax.dot_general(
        A, X, (((1,), (0,)), ((), ())),
        preferred_element_type=jnp.float32,
        precision=jax.lax.Precision.HIGHEST,
    )
    Xq = x_ref[0, pl.ds(qbase, Bq), :]
    mix = params_ref[0]
    scale = params_ref[2]
    blended = mix * Xq + (1.0 - mix) * ctx
    t = blended * gain_ref[...] + bias_ref[...]
    g = 0.5 * t * (1.0 + jax.lax.erf(t * 0.7071067811865476))
    out_ref[0] = g * scale


def kernel(x, gain, bias, log_mix, log_alpha, log_scale):
    B, T, D = x.shape
    Bq = 256
    k_sim = min(K_SIM, T - 1)
    k_con = min(K_CON, max(0, T - 1 - k_sim))
    kp = k_sim + k_con

    mix = jax.nn.sigmoid(log_mix)
    alpha = jax.nn.sigmoid(log_alpha)
    scale = jax.nn.softplus(log_scale) + 0.01
    params = jnp.stack([mix, alpha, scale]).astype(jnp.float32)

    xn = pl.pallas_call(
        _norm_body,
        grid=(B * T // Bq,),
        in_specs=[pl.BlockSpec((Bq, D), lambda i: (i, 0))],
        out_specs=pl.BlockSpec((Bq, D), lambda i: (i, 0)),
        out_shape=jax.ShapeDtypeStruct((B * T, D), jnp.float32),
    )(x.reshape(B * T, D)).reshape(B, T, D)

    idx, w = pl.pallas_call(
        functools.partial(_picks_body, T=T, Bq=Bq, k_sim=k_sim, k_con=k_con),
        grid=(B, T // Bq),
        in_specs=[
            pl.BlockSpec(memory_space=pltpu.SMEM),
            pl.BlockSpec((1, T, D), lambda b, q: (b, 0, 0)),
        ],
        out_specs=[
            pl.BlockSpec((1, Bq, kp), lambda b, q: (b, q, 0)),
            pl.BlockSpec((1, Bq, kp), lambda b, q: (b, q, 0)),
        ],
        out_shape=[
            jax.ShapeDtypeStruct((B, T, kp), jnp.int32),
            jax.ShapeDtypeStruct((B, T, kp), jnp.float32),
        ],
    )(params, xn)

    delta = pl.pallas_call(
        functools.partial(_agg_body, T=T, Bq=Bq, kp=kp),
        grid=(B, T // Bq),
        in_specs=[
            pl.BlockSpec(memory_space=pltpu.SMEM),
            pl.BlockSpec((1, D), lambda b, q: (0, 0)),
            pl.BlockSpec((1, D), lambda b, q: (0, 0)),
            pl.BlockSpec((1, T, D), lambda b, q: (b, 0, 0)),
            pl.BlockSpec((1, Bq, kp), lambda b, q: (b, q, 0)),
            pl.BlockSpec((1, Bq, kp), lambda b, q: (b, q, 0)),
        ],
        out_specs=pl.BlockSpec((1, Bq, D), lambda b, q: (b, q, 0)),
        out_shape=jax.ShapeDtypeStruct((B, T, D), jnp.float32),
    )(params, gain.reshape(1, D), bias.reshape(1, D), x, idx, w)

    return delta


# speedup vs baseline: 11.3766x; 1.4380x over previous
import functools
import jax
import jax.numpy as jnp
from jax.experimental import pallas as pl
from jax.experimental.pallas import tpu as pltpu

K_SIM = 8
K_CON = 4
NEG_BIG = -1.0e9
POS_BIG = 1.0e9
INVALID_THRESH = -0.5e9


def _norm_body(x_ref, xn_ref):
    x = x_ref[...]
    n = jnp.sqrt(jnp.sum(x * x, axis=-1, keepdims=True))
    xn_ref[...] = x / jnp.maximum(n, 1e-12)


def _picks_body(params_ref, xn_ref, idx_ref, w_ref, sim_ref, *, T, Bq,
                k_sim, k_con):
    qi = pl.program_id(1)
    qbase = qi * Bq
    Xnq = xn_ref[0, pl.ds(qbase, Bq), :].astype(jnp.bfloat16)
    nkb = T // Bq
    for kb in range(nkb):
        @pl.when(kb <= qi)
        def _(kb=kb):
            Xk = xn_ref[0, pl.ds(kb * Bq, Bq), :].astype(jnp.bfloat16)
            sim_ref[:, kb * Bq:(kb + 1) * Bq] = jax.lax.dot_general(
                Xnq, Xk, (((1,), (1,)), ((), ())),
                preferred_element_type=jnp.float32,
            )
    rows = qbase + jax.lax.broadcasted_iota(jnp.int32, (Bq, 1), 0)
    cols = jax.lax.broadcasted_iota(jnp.int32, (Bq, T), 1)
    valid = cols < rows
    work = jnp.where(valid, sim_ref[...], NEG_BIG)

    alpha = params_ref[1]

    idxs = []
    oks = []
    for _ in range(k_sim):
        m = jnp.max(work, axis=1, keepdims=True)
        hit = work == m
        idx = jnp.min(jnp.where(hit, cols, T), axis=1, keepdims=True)
        pick = cols == idx
        ok = m > INVALID_THRESH
        idxs.append(idx)
        oks.append(ok)
        work = jnp.where(pick, 3.0 * NEG_BIG, work)

    deg_sim = functools.reduce(
        lambda a, b: a + b, [ok.astype(jnp.float32) for ok in oks[:k_sim]]
    ) if k_sim else jnp.zeros((Bq, 1), jnp.float32)

    simc = jnp.where(work > INVALID_THRESH, work, POS_BIG)
    mcon = jnp.maximum(0, k_con - (T - rows))
    oks_c = []
    for j in range(k_con):
        mn = jnp.min(simc, axis=1, keepdims=True)
        hit = simc == mn
        idx = jnp.min(jnp.where(hit, cols, T), axis=1, keepdims=True)
        pick = cols == idx
        ok = (mn < -INVALID_THRESH) & (j < mcon)
        idxs.append(idx)
        oks_c.append(ok)
        simc = jnp.where(pick, POS_BIG, simc)

    deg_con = functools.reduce(
        lambda a, b: a + b, [ok.astype(jnp.float32) for ok in oks_c]
    ) if k_con else jnp.zeros((Bq, 1), jnp.float32)

    w_sim = alpha / jnp.maximum(deg_sim, 1.0)
    w_con = (1.0 - alpha) / jnp.maximum(deg_con, 1.0)
    ws = [jnp.where(ok, w_sim, 0.0) for ok in oks]
    ws += [jnp.where(ok, w_con, 0.0) for ok in oks_c]

    idx_ref[0] = jnp.concatenate(idxs, axis=1)
    w_ref[0] = jnp.concatenate(ws, axis=1)


def _agg_body(params_ref, gain_ref, bias_ref, x_ref, idx_ref, w_ref, out_ref,
              acc_ref, *, T, Bq, kp):
    qi = pl.program_id(1)
    qbase = qi * Bq
    idx = idx_ref[0]
    w = w_ref[0]
    cols = jax.lax.broadcasted_iota(jnp.int32, (Bq, T), 1)
    acc_ref[...] = jnp.zeros((Bq, x_ref.shape[2]), jnp.float32)
    nkb = T // Bq
    for kb in range(nkb):
        @pl.when(kb <= qi)
        def _(kb=kb):
            ck = cols[:, kb * Bq:(kb + 1) * Bq]
            A = jnp.zeros((Bq, Bq), jnp.float32)
            for k in range(kp):
                A = A + jnp.where(ck == idx[:, k:k + 1], w[:, k:k + 1], 0.0)
            Xk = x_ref[0, pl.ds(kb * Bq, Bq), :].astype(jnp.bfloat16)
            acc_ref[...] += jax.lax.dot_general(
                A.astype(jnp.bfloat16), Xk, (((1,), (0,)), ((), ())),
                preferred_element_type=jnp.float32,
            )
    ctx = acc_ref[...]
    Xq = x_ref[0, pl.ds(qbase, Bq), :]
    mix = params_ref[0]
    scale = params_ref[2]
    blended = mix * Xq + (1.0 - mix) * ctx
    t = blended * gain_ref[...] + bias_ref[...]
    g = 0.5 * t * (1.0 + jax.lax.erf(t * 0.7071067811865476))
    out_ref[0] = g * scale


def kernel(x, gain, bias, log_mix, log_alpha, log_scale):
    B, T, D = x.shape
    Bq = 256
    k_sim = min(K_SIM, T - 1)
    k_con = min(K_CON, max(0, T - 1 - k_sim))
    kp = k_sim + k_con

    mix = jax.nn.sigmoid(log_mix)
    alpha = jax.nn.sigmoid(log_alpha)
    scale = jax.nn.softplus(log_scale) + 0.01
    params = jnp.stack([mix, alpha, scale]).astype(jnp.float32)

    xn = pl.pallas_call(
        _norm_body,
        grid=(B * T // Bq,),
        in_specs=[pl.BlockSpec((Bq, D), lambda i: (i, 0))],
        out_specs=pl.BlockSpec((Bq, D), lambda i: (i, 0)),
        out_shape=jax.ShapeDtypeStruct((B * T, D), jnp.float32),
    )(x.reshape(B * T, D)).reshape(B, T, D)

    idx, w = pl.pallas_call(
        functools.partial(_picks_body, T=T, Bq=Bq, k_sim=k_sim, k_con=k_con),
        grid=(B, T // Bq),
        in_specs=[
            pl.BlockSpec(memory_space=pltpu.SMEM),
            pl.BlockSpec((1, T, D), lambda b, q: (b, 0, 0)),
        ],
        out_specs=[
            pl.BlockSpec((1, Bq, kp), lambda b, q: (b, q, 0)),
            pl.BlockSpec((1, Bq, kp), lambda b, q: (b, q, 0)),
        ],
        out_shape=[
            jax.ShapeDtypeStruct((B, T, kp), jnp.int32),
            jax.ShapeDtypeStruct((B, T, kp), jnp.float32),
        ],
        scratch_shapes=[pltpu.VMEM((Bq, T), jnp.float32)],
    )(params, xn)

    delta = pl.pallas_call(
        functools.partial(_agg_body, T=T, Bq=Bq, kp=kp),
        grid=(B, T // Bq),
        in_specs=[
            pl.BlockSpec(memory_space=pltpu.SMEM),
            pl.BlockSpec((1, D), lambda b, q: (0, 0)),
            pl.BlockSpec((1, D), lambda b, q: (0, 0)),
            pl.BlockSpec((1, T, D), lambda b, q: (b, 0, 0)),
            pl.BlockSpec((1, Bq, kp), lambda b, q: (b, q, 0)),
            pl.BlockSpec((1, Bq, kp), lambda b, q: (b, q, 0)),
        ],
        out_specs=pl.BlockSpec((1, Bq, D), lambda b, q: (b, q, 0)),
        out_shape=jax.ShapeDtypeStruct((B, T, D), jnp.float32),
        scratch_shapes=[pltpu.VMEM((Bq, D), jnp.float32)],
    )(params, gain.reshape(1, D), bias.reshape(1, D), x, idx, w)

    return delta


# speedup vs baseline: 21.1824x; 1.8619x over previous
import functools
import jax
import jax.numpy as jnp
from jax.experimental import pallas as pl
from jax.experimental.pallas import tpu as pltpu

K_SIM = 8
K_CON = 4
NEG_BIG = -1.0e9
POS_BIG = 1.0e9
KILL_NEG = -3.0e9
KILL_POS = 3.0e9
INVALID_THRESH = -0.5e9


def _norm_body(x_ref, xn_ref, xb_ref):
    x = x_ref[...]
    n = jnp.sqrt(jnp.sum(x * x, axis=-1, keepdims=True))
    xn_ref[...] = (x / jnp.maximum(n, 1e-12)).astype(jnp.bfloat16)
    xb_ref[...] = x.astype(jnp.bfloat16)


def _main_body(params_ref, gain_ref, bias_ref, xn_ref, xb_ref, xq_ref,
               out_ref, w_ref, acc_ref, *, T, Bq, k_sim, k_con):
    qi = pl.program_id(1)
    nkb = T // Bq
    qbase = qi * Bq
    D = xb_ref.shape[2]

    Xnq = xn_ref[0, pl.ds(qbase, Bq), :]
    for kb in range(nkb):
        @pl.when(kb <= qi)
        def _(kb=kb):
            Xk = xn_ref[0, pl.ds(kb * Bq, Bq), :]
            w_ref[:, kb * Bq:(kb + 1) * Bq] = jax.lax.dot_general(
                Xnq, Xk, (((1,), (1,)), ((), ())),
                preferred_element_type=jnp.float32,
            )
    rows = qbase + jax.lax.broadcasted_iota(jnp.int32, (Bq, 1), 0)
    cols = jax.lax.broadcasted_iota(jnp.int32, (Bq, T), 1)
    valid = cols < rows
    work = jnp.where(valid, w_ref[...], NEG_BIG)

    alpha = params_ref[1]

    deg_sim = jnp.zeros((Bq, 1), jnp.float32)
    for _ in range(k_sim):
        m = jnp.max(work, axis=1, keepdims=True)
        deg_sim += (m > INVALID_THRESH).astype(jnp.float32)
        work = jnp.where(work == m, KILL_NEG, work)
    m_sim = (work == KILL_NEG) & valid
    w_sim = alpha / jnp.maximum(deg_sim, 1.0)
    w_ref[...] = jnp.where(m_sim, w_sim, 0.0)

    if k_con > 0:
        @pl.when(qi == nkb - 1)
        def _():
            simc = jnp.where(work > INVALID_THRESH, work, POS_BIG)
            mcon = jnp.maximum(0, k_con - (T - rows))
            m_con = jnp.zeros((Bq, T), jnp.bool_)
            deg_con = jnp.zeros((Bq, 1), jnp.float32)
            sc = simc
            for j in range(k_con):
                mn = jnp.min(sc, axis=1, keepdims=True)
                ok = (mn < -INVALID_THRESH) & (j < mcon)
                hit = sc == mn
                m_con = m_con | (hit & ok)
                deg_con += ok.astype(jnp.float32)
                sc = jnp.where(hit, KILL_POS, sc)
            w_con = (1.0 - alpha) / jnp.maximum(deg_con, 1.0)
            w_ref[...] += jnp.where(m_con, w_con, 0.0)

    acc_ref[...] = jnp.zeros((Bq, D), jnp.float32)
    for kb in range(nkb):
        @pl.when(kb <= qi)
        def _(kb=kb):
            A = w_ref[:, kb * Bq:(kb + 1) * Bq].astype(jnp.bfloat16)
            Xk = xb_ref[0, pl.ds(kb * Bq, Bq), :]
            acc_ref[...] += jax.lax.dot_general(
                A, Xk, (((1,), (0,)), ((), ())),
                preferred_element_type=jnp.float32,
            )

    mix = params_ref[0]
    scale = params_ref[2]
    blended = mix * xq_ref[0] + (1.0 - mix) * acc_ref[...]
    t = blended * gain_ref[...] + bias_ref[...]
    g = 0.5 * t * (1.0 + jax.lax.erf(t * 0.7071067811865476))
    out_ref[0] = g * scale


def kernel(x, gain, bias, log_mix, log_alpha, log_scale):
    B, T, D = x.shape
    Bq = 256
    k_sim = min(K_SIM, T - 1)
    k_con = min(K_CON, max(0, T - 1 - k_sim))

    mix = jax.nn.sigmoid(log_mix)
    alpha = jax.nn.sigmoid(log_alpha)
    scale = jax.nn.softplus(log_scale) + 0.01
    params = jnp.stack([mix, alpha, scale]).astype(jnp.float32)

    xn, xb = pl.pallas_call(
        _norm_body,
        grid=(B * T // Bq,),
        in_specs=[pl.BlockSpec((Bq, D), lambda i: (i, 0))],
        out_specs=[
            pl.BlockSpec((Bq, D), lambda i: (i, 0)),
            pl.BlockSpec((Bq, D), lambda i: (i, 0)),
        ],
        out_shape=[
            jax.ShapeDtypeStruct((B * T, D), jnp.bfloat16),
            jax.ShapeDtypeStruct((B * T, D), jnp.bfloat16),
        ],
    )(x.reshape(B * T, D))
    xn = xn.reshape(B, T, D)
    xb = xb.reshape(B, T, D)

    delta = pl.pallas_call(
        functools.partial(_main_body, T=T, Bq=Bq, k_sim=k_sim, k_con=k_con),
        grid=(B, T // Bq),
        in_specs=[
            pl.BlockSpec(memory_space=pltpu.SMEM),
            pl.BlockSpec((1, D), lambda b, q: (0, 0)),
            pl.BlockSpec((1, D), lambda b, q: (0, 0)),
            pl.BlockSpec((1, T, D), lambda b, q: (b, 0, 0)),
            pl.BlockSpec((1, T, D), lambda b, q: (b, 0, 0)),
            pl.BlockSpec((1, Bq, D), lambda b, q: (b, q, 0)),
        ],
        out_specs=pl.BlockSpec((1, Bq, D), lambda b, q: (b, q, 0)),
        out_shape=jax.ShapeDtypeStruct((B, T, D), jnp.float32),
        scratch_shapes=[
            pltpu.VMEM((Bq, T), jnp.float32),
            pltpu.VMEM((Bq, D), jnp.float32),
        ],
    )(params, gain.reshape(1, D), bias.reshape(1, D), xn, xb, x)

    return delta
